# baseline (speedup 1.0000x reference)
import functools

import jax
import jax.numpy as jnp
from jax import lax
from jax.experimental import pallas as pl
from jax.experimental.pallas import tpu as pltpu
from jax.experimental.pallas import tpu_sc as plsc

_K = 512
_ALPHA = 0.01
_GAMMA = 1.0

_D = 32768
_B = 128
_NW = 32
_RPW = _B // _NW

_ORDER = list(range(7, -1, -1)) + list(range(8, 16))


def _select_kth_raw(rowbuf, cand, hist, lanes):
    ones = jnp.ones((16,), jnp.int32)
    zeros16 = jnp.zeros((16,), jnp.int32)
    k_rem = jnp.full((16,), _K, jnp.int32)
    n_cand = jnp.int32(_D)
    prefix = zeros16
    flip_v = zeros16

    for level in range(4):
        shift = 24 - 8 * level

        @plsc.parallel_loop(0, 256, 1, unroll=8)
        def _(i):
            hist[pl.ds(i * 16, 16)] = zeros16

        if level == 0:
            @plsc.parallel_loop(0, _D // 16, 1, unroll=8)
            def _(i):
                digit = (rowbuf[pl.ds(i * 16, 16)] >> shift) & 0xFF
                plsc.addupdate_scatter(hist, [lanes * 256 + digit], ones)
        else:
            nv = (n_cand + 15) >> 4
            n_cand_s = n_cand
            flip_s = flip_v

            @plsc.parallel_loop(0, nv, 1, unroll=4)
            def _(i):
                v = cand[pl.ds(i * 16, 16)]
                valid = (i * 16 + lanes) < n_cand_s
                digit = ((v >> shift) & 0xFF) ^ flip_s
                plsc.addupdate_scatter(hist, [lanes * 256 + digit], ones,
                                       mask=valid)

        def _chunk_tot(c):
            tot = zeros16
            for s in range(16):
                tot = tot + hist[pl.ds(s * 256 + c * 16, 16)]
            return tot

        if level == 0:
            carry = zeros16
            found = jnp.zeros((16,), jnp.bool_)
            dwin = zeros16
            cntgt = zeros16
            for c in _ORDER:
                tot = _chunk_tot(c)
                if c < 8:
                    g = lax.rev(jnp.cumsum(lax.rev(tot, (0,))), (0,)) + carry
                    npop = plsc.all_reduce_population_count(g >= k_rem)
                    jstar = npop - 1
                    s_gt = jnp.sum(jnp.where(lanes > jstar, tot, 0))
                else:
                    g = jnp.cumsum(tot) + carry
                    npop = plsc.all_reduce_population_count(g >= k_rem)
                    jstar = 16 - npop
                    s_gt = jnp.sum(jnp.where(lanes < jstar, tot, 0))
                upd = (npop > 0) & jnp.logical_not(found)
                dwin = jnp.where(upd, c * 16 + jstar, dwin)
                cntgt = jnp.where(upd, carry + s_gt, cntgt)
                found = found | (npop > 0)
                carry = carry + jnp.sum(tot)
            flip_v = jnp.where(dwin >= 128, jnp.int32(0xFF), jnp.int32(0))
            prefix = dwin << shift
        else:
            def walk(i, st):
                carry, found, dwin, cntgt = st
                c = 15 - i
                tot = _chunk_tot(c)
                g = lax.rev(jnp.cumsum(lax.rev(tot, (0,))), (0,)) + carry
                npop = plsc.all_reduce_population_count(g >= k_rem)
                jstar = npop - 1
                s_gt = jnp.sum(jnp.where(lanes > jstar, tot, 0))
                upd = (npop > 0) & jnp.logical_not(found)
                dwin = jnp.where(upd, c * 16 + jstar, dwin)
                cntgt = jnp.where(upd, carry + s_gt, cntgt)
                found = found | (npop > 0)
                carry = carry + jnp.sum(tot)
                return carry, found, dwin, cntgt

            init = (zeros16, jnp.zeros((16,), jnp.bool_), zeros16, zeros16)
            _, _, dwin, cntgt = lax.fori_loop(0, 16, walk, init)
            prefix = prefix | ((dwin ^ flip_v) << shift)

        k_rem = k_rem - cntgt

        if level < 3:
            if level == 0:
                dwin_s = dwin

                @plsc.parallel_loop(0, _D // 16, 1, unroll=8,
                                    carry=jnp.int32(0))
                def n_cand(i, off):
                    v = rowbuf[pl.ds(i * 16, 16)]
                    m = ((v >> shift) & 0xFF) == dwin_s
                    plsc.store_compressed(cand.at[pl.ds(off, 16)], v,
                                          mask=m)
                    return off + jnp.sum(m.astype(jnp.int32))
            else:
                nv = (n_cand + 15) >> 4
                n_cand_s = n_cand
                flip_s = flip_v
                dwin_s = dwin

                @plsc.parallel_loop(0, nv, 1, unroll=4,
                                    carry=jnp.int32(0))
                def n_cand(i, off):
                    v = cand[pl.ds(i * 16, 16)]
                    valid = (i * 16 + lanes) < n_cand_s
                    m = valid & ((((v >> shift) & 0xFF) ^ flip_s) == dwin_s)
                    plsc.store_compressed(cand.at[pl.ds(off, 16)], v,
                                          mask=m)
                    return off + jnp.sum(m.astype(jnp.int32))
    return prefix


def _thr_body(x_hbm, thr_hbm, rowbuf0, rowbuf1, cand, hist, thrv, sems):
    wid = lax.axis_index("s") * 2 + lax.axis_index("c")
    lanes = lax.iota(jnp.int32, 16)
    thr_acc = jnp.zeros((16,), jnp.int32)
    bufs = [rowbuf0, rowbuf1]
    copies = [pltpu.async_copy(x_hbm.at[wid * _RPW], rowbuf0, sems.at[0])]
    for r in range(_RPW):
        if r + 1 < _RPW:
            copies.append(pltpu.async_copy(x_hbm.at[wid * _RPW + r + 1],
                                           bufs[(r + 1) % 2],
                                           sems.at[(r + 1) % 2]))
        copies[r].wait()
        bits = _select_kth_raw(bufs[r % 2], cand, hist, lanes)
        thr_acc = jnp.where(lanes == r, bits, thr_acc)
    thrv[...] = thr_acc
    pltpu.sync_copy(thrv, thr_hbm.at[wid])


_thr_sc = functools.partial(
    pl.kernel,
    out_type=jax.ShapeDtypeStruct((_NW, 16), jnp.int32),
    mesh=plsc.VectorSubcoreMesh(core_axis_name="c", subcore_axis_name="s",
                                num_cores=2, num_subcores=16),
    compiler_params=pltpu.CompilerParams(needs_layout_passes=False),
    scratch_types=[
        pltpu.VMEM((_D,), jnp.int32),
        pltpu.VMEM((_D,), jnp.int32),
        pltpu.VMEM((_D + 16,), jnp.int32),
        pltpu.VMEM((4096,), jnp.int32),
        pltpu.VMEM((16,), jnp.int32),
        pltpu.SemaphoreType.DMA((2,)),
    ],
)(_thr_body)


def _out_kernel(x_ref, thr_ref, duty_ref, out_ref):
    x = x_ref[...]
    thr_f = lax.bitcast_convert_type(thr_ref[...], jnp.float32)
    mask = x >= thr_f
    cc = jnp.sum(mask.astype(jnp.float32), axis=0, keepdims=True)
    duty_new = duty_ref[...] * (1.0 - _ALPHA) + (_ALPHA / x.shape[0]) * cc
    boost = jnp.exp(-_GAMMA * (duty_new - _K / _D))
    out_ref[...] = jnp.where(mask, x * boost, 0.0)


def kernel(x, duty):
    b, d = x.shape
    xi = lax.bitcast_convert_type(x, jnp.int32)
    thr_packed = _thr_sc(xi)
    thr = thr_packed[:, :_RPW].reshape(b, 1)

    cb = 8192
    out = pl.pallas_call(
        _out_kernel,
        grid=(d // cb,),
        in_specs=[
            pl.BlockSpec((b, cb), lambda j: (0, j)),
            pl.BlockSpec((b, 1), lambda j: (0, 0)),
            pl.BlockSpec((1, cb), lambda j: (0, j)),
        ],
        out_specs=pl.BlockSpec((b, cb), lambda j: (0, j)),
        out_shape=jax.ShapeDtypeStruct((b, d), jnp.float32),
    )(x, thr, duty)
    return out


# speedup vs baseline: 1.0814x; 1.0814x over previous
import functools

import jax
import jax.numpy as jnp
from jax import lax
from jax.experimental import pallas as pl
from jax.experimental.pallas import tpu as pltpu
from jax.experimental.pallas import tpu_sc as plsc

_K = 512
_ALPHA = 0.01
_GAMMA = 1.0

_D = 32768
_B = 128
_NW = 32
_RPW = _B // _NW

_ORDER = list(range(7, -1, -1)) + list(range(8, 16))


def _select_kth_raw(rowbuf, cand, hist, lanes):
    ones = jnp.ones((16,), jnp.int32)
    zeros16 = jnp.zeros((16,), jnp.int32)
    k_rem = jnp.full((16,), _K, jnp.int32)
    n_cand = jnp.int32(_D)
    prefix = zeros16
    flip_v = zeros16

    for level in range(4):
        shift = 24 - 8 * level

        @plsc.parallel_loop(0, 256, 1, unroll=8)
        def _(i):
            hist[pl.ds(i * 16, 16)] = zeros16

        if level == 0:
            @plsc.parallel_loop(0, _D // 16, 1, unroll=8)
            def _(i):
                s = lax.bitcast_convert_type(rowbuf[pl.ds(i * 16, 16)],
                                             jnp.int32)
                digit = (s >> shift) & 0xFF
                plsc.addupdate_scatter(hist, [lanes * 256 + digit], ones)
        else:
            nv = (n_cand + 15) >> 4
            n_cand_s = n_cand
            flip_s = flip_v

            @plsc.parallel_loop(0, nv, 1, unroll=4)
            def _(i):
                v = cand[pl.ds(i * 16, 16)]
                valid = (i * 16 + lanes) < n_cand_s
                digit = ((v >> shift) & 0xFF) ^ flip_s
                plsc.addupdate_scatter(hist, [lanes * 256 + digit], ones,
                                       mask=valid)

        def _chunk_tot(c):
            tot = zeros16
            for s in range(16):
                tot = tot + hist[pl.ds(s * 256 + c * 16, 16)]
            return tot

        if level == 0:
            def walk0(i, st):
                carry, found, dwin, cntgt = st
                c = jnp.where(i < 8, 7 - i, i)
                neg = i >= 8
                tot = _chunk_tot(c)
                pre = jnp.cumsum(tot)
                total = jnp.sum(tot)
                g = jnp.where(neg, pre, total - pre + tot) + carry
                npop = plsc.all_reduce_population_count(g >= k_rem)
                jstar = jnp.where(neg, 16 - npop, npop - 1)
                m_gt = jnp.where(neg, lanes < jstar, lanes > jstar)
                s_gt = jnp.sum(jnp.where(m_gt, tot, 0))
                upd = (npop > 0) & jnp.logical_not(found)
                dwin = jnp.where(upd, c * 16 + jstar, dwin)
                cntgt = jnp.where(upd, carry + s_gt, cntgt)
                found = found | (npop > 0)
                return carry + total, found, dwin, cntgt

            init = (zeros16, jnp.zeros((16,), jnp.bool_), zeros16, zeros16)
            _, _, dwin, cntgt = lax.fori_loop(0, 16, walk0, init)
            flip_v = jnp.where(dwin >= 128, jnp.int32(0xFF), jnp.int32(0))
            prefix = dwin << shift
        else:
            def walk(i, st):
                carry, found, dwin, cntgt = st
                c = 15 - i
                tot = _chunk_tot(c)
                g = lax.rev(jnp.cumsum(lax.rev(tot, (0,))), (0,)) + carry
                npop = plsc.all_reduce_population_count(g >= k_rem)
                jstar = npop - 1
                s_gt = jnp.sum(jnp.where(lanes > jstar, tot, 0))
                upd = (npop > 0) & jnp.logical_not(found)
                dwin = jnp.where(upd, c * 16 + jstar, dwin)
                cntgt = jnp.where(upd, carry + s_gt, cntgt)
                found = found | (npop > 0)
                carry = carry + jnp.sum(tot)
                return carry, found, dwin, cntgt

            init = (zeros16, jnp.zeros((16,), jnp.bool_), zeros16, zeros16)
            _, _, dwin, cntgt = lax.fori_loop(0, 16, walk, init)
            prefix = prefix | ((dwin ^ flip_v) << shift)

        k_rem = k_rem - cntgt

        if level < 3:
            if level == 0:
                dwin_s = dwin

                @plsc.parallel_loop(0, _D // 16, 1, unroll=8,
                                    carry=jnp.int32(0))
                def n_cand(i, off):
                    v = lax.bitcast_convert_type(rowbuf[pl.ds(i * 16, 16)],
                                                 jnp.int32)
                    m = ((v >> shift) & 0xFF) == dwin_s
                    plsc.store_compressed(cand.at[pl.ds(off, 16)], v,
                                          mask=m)
                    return off + jnp.sum(m.astype(jnp.int32))
            else:
                nv = (n_cand + 15) >> 4
                n_cand_s = n_cand
                flip_s = flip_v
                dwin_s = dwin

                @plsc.parallel_loop(0, nv, 1, unroll=4,
                                    carry=jnp.int32(0))
                def n_cand(i, off):
                    v = cand[pl.ds(i * 16, 16)]
                    valid = (i * 16 + lanes) < n_cand_s
                    m = valid & ((((v >> shift) & 0xFF) ^ flip_s) == dwin_s)
                    plsc.store_compressed(cand.at[pl.ds(off, 16)], v,
                                          mask=m)
                    return off + jnp.sum(m.astype(jnp.int32))
    return prefix


def _thr_body(x_hbm, thr_hbm, rowbuf0, rowbuf1, cand, hist, thrv, sems):
    wid = lax.axis_index("s") * 2 + lax.axis_index("c")
    lanes = lax.iota(jnp.int32, 16)
    thr_acc = jnp.zeros((16,), jnp.int32)
    bufs = [rowbuf0, rowbuf1]
    copies = [pltpu.async_copy(x_hbm.at[wid * _RPW], rowbuf0, sems.at[0])]
    for r in range(_RPW):
        if r + 1 < _RPW:
            copies.append(pltpu.async_copy(x_hbm.at[wid * _RPW + r + 1],
                                           bufs[(r + 1) % 2],
                                           sems.at[(r + 1) % 2]))
        copies[r].wait()
        bits = _select_kth_raw(bufs[r % 2], cand, hist, lanes)
        thr_acc = jnp.where(lanes == r, bits, thr_acc)
    thrv[...] = thr_acc
    pltpu.sync_copy(thrv, thr_hbm.at[wid])


_thr_sc = functools.partial(
    pl.kernel,
    out_type=jax.ShapeDtypeStruct((_NW, 16), jnp.int32),
    mesh=plsc.VectorSubcoreMesh(core_axis_name="c", subcore_axis_name="s",
                                num_cores=2, num_subcores=16),
    compiler_params=pltpu.CompilerParams(needs_layout_passes=False),
    scratch_types=[
        pltpu.VMEM((_D,), jnp.float32),
        pltpu.VMEM((_D,), jnp.float32),
        pltpu.VMEM((_D + 16,), jnp.int32),
        pltpu.VMEM((4096,), jnp.int32),
        pltpu.VMEM((16,), jnp.int32),
        pltpu.SemaphoreType.DMA((2,)),
    ],
)(_thr_body)


def _out_kernel(x_ref, thr_ref, duty_ref, out_ref):
    x = x_ref[...]
    thr_f = lax.bitcast_convert_type(thr_ref[...], jnp.float32)
    mask = x >= thr_f
    cc = jnp.sum(mask.astype(jnp.float32), axis=0, keepdims=True)
    duty_new = duty_ref[...] * (1.0 - _ALPHA) + (_ALPHA / x.shape[0]) * cc
    boost = jnp.exp(-_GAMMA * (duty_new - _K / _D))
    out_ref[...] = jnp.where(mask, x * boost, 0.0)


def kernel(x, duty):
    b, d = x.shape
    thr_packed = _thr_sc(x)
    thr = thr_packed[:, :_RPW].reshape(b, 1)

    cb = 8192
    out = pl.pallas_call(
        _out_kernel,
        grid=(d // cb,),
        in_specs=[
            pl.BlockSpec((b, cb), lambda j: (0, j)),
            pl.BlockSpec((b, 1), lambda j: (0, 0)),
            pl.BlockSpec((1, cb), lambda j: (0, j)),
        ],
        out_specs=pl.BlockSpec((b, cb), lambda j: (0, j)),
        out_shape=jax.ShapeDtypeStruct((b, d), jnp.float32),
    )(x, thr, duty)
    return out


# speedup vs baseline: 1.1546x; 1.0677x over previous
import functools

import jax
import jax.numpy as jnp
from jax import lax
from jax.experimental import pallas as pl
from jax.experimental.pallas import tpu as pltpu
from jax.experimental.pallas import tpu_sc as plsc

_K = 512
_ALPHA = 0.01
_GAMMA = 1.0

_D = 32768
_B = 128
_NW = 32
_RPW = _B // _NW


def _sortable(x_f32):
    s = lax.bitcast_convert_type(x_f32, jnp.int32)
    return s ^ ((s >> 31) & jnp.int32(0x7FFFFFFF))


def _select_kth_key(rowbuf, cand, hist, lanes):
    ones = jnp.ones((16,), jnp.int32)
    zeros16 = jnp.zeros((16,), jnp.int32)
    k_rem = jnp.full((16,), _K, jnp.int32)
    prefix = zeros16
    n_cand = jnp.int32(_D)

    for level in range(4):
        shift = 24 - 8 * level
        flip = 0x80 if level == 0 else 0

        @plsc.parallel_loop(0, 256, 1, unroll=8)
        def _(i):
            hist[pl.ds(i * 16, 16)] = zeros16

        if level == 0:
            @plsc.parallel_loop(0, _D // 16, 1, unroll=8)
            def _(i):
                key = _sortable(rowbuf[pl.ds(i * 16, 16)])
                digit = ((key >> shift) & 0xFF) ^ flip
                plsc.addupdate_scatter(hist, [lanes * 256 + digit], ones)
        else:
            nv = (n_cand + 15) >> 4
            n_cand_s = n_cand

            @plsc.parallel_loop(0, nv, 1, unroll=4)
            def _(i):
                key = cand[pl.ds(i * 16, 16)]
                valid = (i * 16 + lanes) < n_cand_s
                digit = ((key >> shift) & 0xFF) ^ flip
                plsc.addupdate_scatter(hist, [lanes * 256 + digit], ones,
                                       mask=valid)

        def walk(i, st):
            carry, found, dwin, cntgt = st
            c = 15 - i
            tot = zeros16
            for s in range(16):
                tot = tot + hist[pl.ds(s * 256 + c * 16, 16)]
            suf = lax.rev(jnp.cumsum(lax.rev(tot, (0,))), (0,))
            g = suf + carry
            in_mask = g >= k_rem
            cnt = plsc.all_reduce_population_count(in_mask)
            jstar = cnt - 1
            s_gt = jnp.sum(jnp.where(lanes > jstar, tot, 0))
            has = cnt > 0
            upd = has & jnp.logical_not(found)
            dwin = jnp.where(upd, c * 16 + jstar, dwin)
            cntgt = jnp.where(upd, carry + s_gt, cntgt)
            found = found | has
            carry = carry + jnp.sum(tot)
            return carry, found, dwin, cntgt

        init = (zeros16, jnp.zeros((16,), jnp.bool_), zeros16, zeros16)
        _, _, dwin, cntgt = lax.fori_loop(0, 16, walk, init)

        prefix = prefix | ((dwin ^ flip) << shift)
        k_rem = k_rem - cntgt

        if level < 3:
            if level == 0:
                @plsc.parallel_loop(0, _D // 16, 1, unroll=8,
                                    carry=jnp.int32(0))
                def n_cand(i, off):
                    key = _sortable(rowbuf[pl.ds(i * 16, 16)])
                    m = (((key >> shift) & 0xFF) ^ flip) == dwin
                    plsc.store_compressed(cand.at[pl.ds(off, 16)], key,
                                          mask=m)
                    return off + jnp.sum(m.astype(jnp.int32))
            else:
                nv = (n_cand + 15) >> 4
                n_cand_s = n_cand

                @plsc.parallel_loop(0, nv, 1, unroll=4,
                                    carry=jnp.int32(0))
                def n_cand(i, off):
                    key = cand[pl.ds(i * 16, 16)]
                    valid = (i * 16 + lanes) < n_cand_s
                    m = valid & ((((key >> shift) & 0xFF) ^ flip) == dwin)
                    plsc.store_compressed(cand.at[pl.ds(off, 16)], key,
                                          mask=m)
                    return off + jnp.sum(m.astype(jnp.int32))
    return prefix


def _thr_body(x_hbm, thr_hbm, rowbuf, cand, hist, thrv, sem):
    wid = lax.axis_index("s") * 2 + lax.axis_index("c")
    lanes = lax.iota(jnp.int32, 16)
    thr_acc = jnp.zeros((16,), jnp.int32)
    for r in range(_RPW):
        pltpu.async_copy(x_hbm.at[wid * _RPW + r], rowbuf, sem).wait()
        key = _select_kth_key(rowbuf, cand, hist, lanes)
        thr_acc = jnp.where(lanes == r, key, thr_acc)
    thrv[...] = thr_acc
    pltpu.sync_copy(thrv, thr_hbm.at[wid])


_thr_sc = functools.partial(
    pl.kernel,
    out_type=jax.ShapeDtypeStruct((_NW, 16), jnp.int32),
    mesh=plsc.VectorSubcoreMesh(core_axis_name="c", subcore_axis_name="s",
                                num_cores=2, num_subcores=16),
    compiler_params=pltpu.CompilerParams(needs_layout_passes=False),
    scratch_types=[
        pltpu.VMEM((_D,), jnp.float32),
        pltpu.VMEM((_D + 16,), jnp.int32),
        pltpu.VMEM((4096,), jnp.int32),
        pltpu.VMEM((16,), jnp.int32),
        pltpu.SemaphoreType.DMA,
    ],
)(_thr_body)


def _out_kernel(x_ref, thr_ref, duty_ref, out_ref):
    x = x_ref[...]
    t = thr_ref[...]
    thr_f = lax.bitcast_convert_type(
        t ^ ((t >> 31) & jnp.int32(0x7FFFFFFF)), jnp.float32)
    mask = x >= thr_f
    cc = jnp.sum(mask.astype(jnp.float32), axis=0, keepdims=True)
    duty_new = duty_ref[...] * (1.0 - _ALPHA) + (_ALPHA / x.shape[0]) * cc
    boost = jnp.exp(-_GAMMA * (duty_new - _K / _D))
    out_ref[...] = jnp.where(mask, x * boost, 0.0)


def kernel(x, duty):
    b, d = x.shape
    thr_packed = _thr_sc(x)
    thr = thr_packed[:, :_RPW].reshape(b, 1)

    cb = 8192
    out = pl.pallas_call(
        _out_kernel,
        grid=(d // cb,),
        in_specs=[
            pl.BlockSpec((b, cb), lambda j: (0, j)),
            pl.BlockSpec((b, 1), lambda j: (0, 0)),
            pl.BlockSpec((1, cb), lambda j: (0, j)),
        ],
        out_specs=pl.BlockSpec((b, cb), lambda j: (0, j)),
        out_shape=jax.ShapeDtypeStruct((b, d), jnp.float32),
    )(x, thr, duty)
    return out
